# baseline (speedup 1.0000x reference)
import functools

import jax
import jax.numpy as jnp
import numpy as np
from jax import lax
from jax.experimental import pallas as pl
from jax.experimental.pallas import tpu as pltpu
from jax.experimental.pallas import tpu_sc as plsc

H_IN, W_IN = 480, 854
H_OUT, W_OUT = 800, 1200
NW = 32
RPW = H_IN // NW
WROWS = 24
WSTART_MAX = H_IN - WROWS
NFULL = W_IN // 16
TAILC0 = W_IN - 32
TAILC = W_IN - 16
MBUF = 15 * W_IN + 16

_mesh = plsc.VectorSubcoreMesh(core_axis_name="c", subcore_axis_name="s")


@functools.partial(
    pl.kernel,
    out_type=jax.ShapeDtypeStruct((NW * 16,), jnp.float32),
    mesh=_mesh,
    scratch_types=[
        pltpu.VMEM((4, WROWS, W_IN), jnp.float32),
        pltpu.VMEM((MBUF,), jnp.float32),
        pltpu.VMEM((MBUF,), jnp.float32),
        pltpu.VMEM((856,), jnp.float32),
        pltpu.VMEM((856,), jnp.float32),
        pltpu.VMEM((16,), jnp.float32),
        pltpu.SemaphoreType.DMA,
    ],
)
def _sc_any_diff(mask_hbm, colmasks_hbm, out_hbm,
                 raw, mxb, mnb, scv, dcv, accv, sem):
    w = lax.axis_index("s") * 2 + lax.axis_index("c")
    row0 = w * RPW
    ws = jnp.minimum((row0 // 8) * 8, WSTART_MAX)
    lr0 = row0 - ws

    copies = [pltpu.async_copy(
        mask_hbm.at[b, 0, pl.ds(ws, WROWS)], raw.at[b], sem)
        for b in range(4)]
    copies.append(pltpu.async_copy(colmasks_hbm.at[pl.ds(0, 856)], scv, sem))
    copies.append(pltpu.async_copy(colmasks_hbm.at[pl.ds(856, 856)], dcv,
                                   sem))
    for c in copies:
        c.wait()

    zeros16 = jnp.zeros((16,), jnp.float32)

    def init_bound(k, x):
        mxb[pl.ds(k * W_IN, 16)] = zeros16
        mnb[pl.ds(k * W_IN, 16)] = zeros16 + 1.0
        return x

    lax.fori_loop(1, 16, init_bound, 0)

    def colf(cc, acc):
        dcf = dcv[pl.ds(cc, 16)]

        def rowf(i, acc):
            lr = lr0 + i
            v0 = raw[0, lr, pl.ds(cc, 16)]
            v1 = raw[1, lr, pl.ds(cc, 16)]
            v2 = raw[2, lr, pl.ds(cc, 16)]
            v3 = raw[3, lr, pl.ds(cc, 16)]
            a = jnp.maximum(jnp.maximum(v0, v1), jnp.maximum(v2, v3))
            b = jnp.minimum(jnp.minimum(v0, v1), jnp.minimum(v2, v3))
            off = i * W_IN + cc
            mxb[pl.ds(off, 16)] = a
            mnb[pl.ds(off, 16)] = b
            a_s = mxb[pl.ds(off + 1, 16)]
            b_s = mnb[pl.ds(off + 1, 16)]
            g = jnp.maximum(a - b, 0.0)
            t2 = dcf * jnp.maximum(jnp.maximum(a, a_s) -
                                   jnp.minimum(b, b_s), 0.0)
            return jnp.maximum(acc, g + t2)

        return lax.fori_loop(0, RPW, rowf, acc)

    acc = colf(TAILC, jnp.zeros((16,), jnp.float32))
    acc = colf(TAILC0, acc)
    acc = lax.fori_loop(
        0, NFULL, lambda c, x: colf((NFULL - 1 - c) * 16, x), acc)

    def col2(cc, acc):
        scf = scv[pl.ds(cc, 16)]
        dcf = dcv[pl.ds(cc, 16)]

        def row2(k, acc):
            off = (1 + 3 * k) * W_IN + cc
            a = mxb[pl.ds(off, 16)]
            b = mnb[pl.ds(off, 16)]
            ad = mxb[pl.ds(off + W_IN, 16)]
            bd = mnb[pl.ds(off + W_IN, 16)]
            a_ds = mxb[pl.ds(off + W_IN + 1, 16)]
            b_ds = mnb[pl.ds(off + W_IN + 1, 16)]
            t3 = scf * jnp.maximum(jnp.maximum(a, ad) -
                                   jnp.minimum(b, bd), 0.0)
            t4 = dcf * jnp.maximum(jnp.maximum(a, a_ds) -
                                   jnp.minimum(b, b_ds), 0.0)
            return jnp.maximum(acc, t3 + t4)

        return lax.fori_loop(0, 5, row2, acc)

    acc = lax.fori_loop(0, NFULL, lambda c, x: col2(c * 16, x), acc)
    acc = col2(TAILC0, acc)
    acc = col2(TAILC, acc)
    accv[pl.ds(0, 16)] = acc
    pltpu.sync_copy(accv, out_hbm.at[pl.ds(w * 16, 16)])


def _tc_reduce_body(x_ref, o_ref):
    o_ref[...] = jnp.max(x_ref[...], axis=0, keepdims=True)


_tc_reduce = pl.pallas_call(
    _tc_reduce_body,
    out_shape=jax.ShapeDtypeStruct((1,), jnp.float32),
)


def _col_masks():
    jj = np.arange(W_OUT)
    ctj = (jj * W_IN) // W_OUT
    cpj = np.clip((W_IN * jj + (W_IN // 2)) // W_OUT, 0, W_IN - 1)
    packed = np.zeros(2 * 856, np.float32)
    np.maximum.at(packed[:W_IN], ctj, (cpj == ctj).astype(np.float32))
    np.maximum.at(packed[856:856 + W_IN], ctj,
                  (cpj == ctj + 1).astype(np.float32))
    return packed


_COL_MASKS = _col_masks()


def kernel(mask):
    partial = _sc_any_diff(mask, _COL_MASKS)
    red = _tc_reduce(partial)
    return red[0] > 0.0


# speedup vs baseline: 1.0057x; 1.0057x over previous
import functools

import jax
import jax.numpy as jnp
import numpy as np
from jax import lax
from jax.experimental import pallas as pl
from jax.experimental.pallas import tpu as pltpu
from jax.experimental.pallas import tpu_sc as plsc

H_IN, W_IN = 480, 854
H_OUT, W_OUT = 800, 1200
NW = 32
RPW = H_IN // NW
WROWS = 24
WSTART_MAX = H_IN - WROWS
NQUAD = 13
TAILC0 = W_IN - 32
TAILC = W_IN - 16
MBUF = 15 * W_IN + 16

_mesh = plsc.VectorSubcoreMesh(core_axis_name="c", subcore_axis_name="s")


@functools.partial(
    pl.kernel,
    out_type=jax.ShapeDtypeStruct((NW * 16,), jnp.float32),
    mesh=_mesh,
    scratch_types=[
        pltpu.VMEM((4, WROWS, W_IN), jnp.float32),
        pltpu.VMEM((MBUF,), jnp.float32),
        pltpu.VMEM((MBUF,), jnp.float32),
        pltpu.VMEM((856,), jnp.float32),
        pltpu.VMEM((856,), jnp.float32),
        pltpu.VMEM((16,), jnp.float32),
        pltpu.SemaphoreType.DMA,
        pltpu.SemaphoreType.DMA,
    ],
)
def _sc_any_diff(mask_hbm, colmasks_hbm, out_hbm,
                 raw, mxb, mnb, scv, dcv, accv, sem, sem2):
    w = lax.axis_index("s") * 2 + lax.axis_index("c")
    row0 = w * RPW
    ws = jnp.minimum((row0 // 8) * 8, WSTART_MAX)
    lr0 = row0 - ws

    copies1 = [pltpu.async_copy(
        mask_hbm.at[b, 0, pl.ds(ws, 16)], raw.at[b, pl.ds(0, 16)], sem)
        for b in range(4)]
    copies2 = [pltpu.async_copy(
        mask_hbm.at[b, 0, pl.ds(ws + 16, 8)], raw.at[b, pl.ds(16, 8)], sem2)
        for b in range(4)]
    copies1.append(pltpu.async_copy(colmasks_hbm.at[pl.ds(0, 856)], scv,
                                    sem))
    copies1.append(pltpu.async_copy(colmasks_hbm.at[pl.ds(856, 856)], dcv,
                                    sem))

    zeros16 = jnp.zeros((16,), jnp.float32)

    def init_bound(k, x):
        mxb[pl.ds(k * W_IN, 16)] = zeros16
        mnb[pl.ds(k * W_IN, 16)] = zeros16 + 1.0
        return x

    lax.fori_loop(1, 16, init_bound, 0)
    for c in copies1:
        c.wait()

    def rowf(i, x):
        lr = lr0 + i
        mbase = i * W_IN

        def chunkf(cc, acc):
            v0 = raw[0, lr, pl.ds(cc, 16)]
            v1 = raw[1, lr, pl.ds(cc, 16)]
            v2 = raw[2, lr, pl.ds(cc, 16)]
            v3 = raw[3, lr, pl.ds(cc, 16)]
            a = jnp.maximum(jnp.maximum(v0, v1), jnp.maximum(v2, v3))
            b = jnp.minimum(jnp.minimum(v0, v1), jnp.minimum(v2, v3))
            off = mbase + cc
            mxb[pl.ds(off, 16)] = a
            mnb[pl.ds(off, 16)] = b
            a_s = mxb[pl.ds(off + 1, 16)]
            b_s = mnb[pl.ds(off + 1, 16)]
            dcf = dcv[pl.ds(cc, 16)]
            g = jnp.maximum(a - b, 0.0)
            t2 = dcf * jnp.maximum(jnp.maximum(a, a_s) -
                                   jnp.minimum(b, b_s), 0.0)
            return jnp.maximum(acc, g + t2)

        acc = chunkf(TAILC, x)
        acc = chunkf(TAILC0, acc)

        def pair(c, acc):
            cc = (2 * NQUAD - 1 - c) * 32
            for s in (16, 0):
                acc = chunkf(cc + s, acc)
            return acc

        return lax.fori_loop(0, 2 * NQUAD, pair, acc)

    n1 = jnp.minimum(16 - lr0, RPW)
    acc = lax.fori_loop(0, n1, rowf, jnp.zeros((16,), jnp.float32))
    for c in copies2:
        c.wait()
    acc = lax.fori_loop(n1, RPW, rowf, acc)

    def row2(k, x):
        base = (1 + 3 * k) * W_IN

        def chunk2(cc, acc):
            off = base + cc
            a = mxb[pl.ds(off, 16)]
            b = mnb[pl.ds(off, 16)]
            ad = mxb[pl.ds(off + W_IN, 16)]
            bd = mnb[pl.ds(off + W_IN, 16)]
            a_ds = mxb[pl.ds(off + W_IN + 1, 16)]
            b_ds = mnb[pl.ds(off + W_IN + 1, 16)]
            scf = scv[pl.ds(cc, 16)]
            dcf = dcv[pl.ds(cc, 16)]
            t3 = scf * jnp.maximum(jnp.maximum(a, ad) -
                                   jnp.minimum(b, bd), 0.0)
            t4 = dcf * jnp.maximum(jnp.maximum(a, a_ds) -
                                   jnp.minimum(b, b_ds), 0.0)
            return jnp.maximum(acc, t3 + t4)

        def pair(c, acc):
            cc = c * 32
            for s in (0, 16):
                acc = chunk2(cc + s, acc)
            return acc

        acc = lax.fori_loop(0, 2 * NQUAD, pair, x)
        acc = chunk2(TAILC0, acc)
        return chunk2(TAILC, acc)

    acc = lax.fori_loop(0, 5, row2, acc)
    accv[pl.ds(0, 16)] = acc
    pltpu.sync_copy(accv, out_hbm.at[pl.ds(w * 16, 16)])


def _tc_reduce_body(x_ref, o_ref):
    o_ref[...] = jnp.max(x_ref[...], axis=0, keepdims=True)


_tc_reduce = pl.pallas_call(
    _tc_reduce_body,
    out_shape=jax.ShapeDtypeStruct((1,), jnp.float32),
)


def _col_masks():
    jj = np.arange(W_OUT)
    ctj = (jj * W_IN) // W_OUT
    cpj = np.clip((W_IN * jj + (W_IN // 2)) // W_OUT, 0, W_IN - 1)
    packed = np.zeros(2 * 856, np.float32)
    np.maximum.at(packed[:W_IN], ctj, (cpj == ctj).astype(np.float32))
    np.maximum.at(packed[856:856 + W_IN], ctj,
                  (cpj == ctj + 1).astype(np.float32))
    return packed


_COL_MASKS = _col_masks()


def kernel(mask):
    partial = _sc_any_diff(mask, _COL_MASKS)
    red = _tc_reduce(partial)
    return red[0] > 0.0


# speedup vs baseline: 1.0181x; 1.0123x over previous
import functools

import jax
import jax.numpy as jnp
import numpy as np
from jax import lax
from jax.experimental import pallas as pl
from jax.experimental.pallas import tpu as pltpu
from jax.experimental.pallas import tpu_sc as plsc

H_IN, W_IN = 480, 854
H_OUT, W_OUT = 800, 1200
NW = 32
RPW = H_IN // NW
WROWS = 24
WSTART_MAX = H_IN - WROWS
NQUAD = 13
TAILC0 = W_IN - 32
TAILC = W_IN - 16
MBUF = 15 * W_IN + 16

_mesh = plsc.VectorSubcoreMesh(core_axis_name="c", subcore_axis_name="s")


@functools.partial(
    pl.kernel,
    out_type=jax.ShapeDtypeStruct((NW * 16,), jnp.float32),
    mesh=_mesh,
    scratch_types=[
        pltpu.VMEM((4, WROWS, W_IN), jnp.float32),
        pltpu.VMEM((MBUF,), jnp.float32),
        pltpu.VMEM((MBUF,), jnp.float32),
        pltpu.VMEM((856,), jnp.float32),
        pltpu.VMEM((856,), jnp.float32),
        pltpu.VMEM((16,), jnp.float32),
        pltpu.SemaphoreType.DMA,
    ],
)
def _sc_any_diff(mask_hbm, colmasks_hbm, out_hbm,
                 raw, mxb, mnb, scv, dcv, accv, sem):
    w = lax.axis_index("s") * 2 + lax.axis_index("c")
    row0 = w * RPW
    ws = jnp.minimum((row0 // 8) * 8, WSTART_MAX)
    lr0 = row0 - ws

    copies = [pltpu.async_copy(
        mask_hbm.at[b, 0, pl.ds(ws, WROWS)], raw.at[b], sem)
        for b in range(4)]
    copies.append(pltpu.async_copy(colmasks_hbm.at[pl.ds(0, 856)], scv, sem))
    copies.append(pltpu.async_copy(colmasks_hbm.at[pl.ds(856, 856)], dcv,
                                   sem))
    for c in copies:
        c.wait()

    zeros16 = jnp.zeros((16,), jnp.float32)

    def init_bound(k, x):
        mxb[pl.ds(k * W_IN, 16)] = zeros16
        mnb[pl.ds(k * W_IN, 16)] = zeros16 + 1.0
        return x

    lax.fori_loop(1, 16, init_bound, 0)

    def rowf(i, x):
        lr = lr0 + i
        mbase = i * W_IN

        def chunkf(cc, acc):
            v0 = raw[0, lr, pl.ds(cc, 16)]
            v1 = raw[1, lr, pl.ds(cc, 16)]
            v2 = raw[2, lr, pl.ds(cc, 16)]
            v3 = raw[3, lr, pl.ds(cc, 16)]
            a = jnp.maximum(jnp.maximum(v0, v1), jnp.maximum(v2, v3))
            b = jnp.minimum(jnp.minimum(v0, v1), jnp.minimum(v2, v3))
            off = mbase + cc
            mxb[pl.ds(off, 16)] = a
            mnb[pl.ds(off, 16)] = b
            a_s = mxb[pl.ds(off + 1, 16)]
            b_s = mnb[pl.ds(off + 1, 16)]
            dcf = dcv[pl.ds(cc, 16)]
            g = jnp.maximum(a - b, 0.0)
            t2 = dcf * jnp.maximum(jnp.maximum(a, a_s) -
                                   jnp.minimum(b, b_s), 0.0)
            return jnp.maximum(acc, g + t2)

        acc = chunkf(TAILC, x)
        acc = chunkf(TAILC0, acc)

        def pair(c, acc):
            cc = (2 * NQUAD - 1 - c) * 32
            for s in (16, 0):
                acc = chunkf(cc + s, acc)
            return acc

        return lax.fori_loop(0, 2 * NQUAD, pair, acc)

    acc = lax.fori_loop(0, RPW, rowf, jnp.zeros((16,), jnp.float32))

    def row2(k, x):
        base = (1 + 3 * k) * W_IN

        def chunk2(cc, acc):
            off = base + cc
            a = mxb[pl.ds(off, 16)]
            b = mnb[pl.ds(off, 16)]
            ad = mxb[pl.ds(off + W_IN, 16)]
            bd = mnb[pl.ds(off + W_IN, 16)]
            a_ds = mxb[pl.ds(off + W_IN + 1, 16)]
            b_ds = mnb[pl.ds(off + W_IN + 1, 16)]
            scf = scv[pl.ds(cc, 16)]
            dcf = dcv[pl.ds(cc, 16)]
            t3 = scf * jnp.maximum(jnp.maximum(a, ad) -
                                   jnp.minimum(b, bd), 0.0)
            t4 = dcf * jnp.maximum(jnp.maximum(a, a_ds) -
                                   jnp.minimum(b, b_ds), 0.0)
            return jnp.maximum(acc, t3 + t4)

        def pair(c, acc):
            cc = c * 32
            for s in (0, 16):
                acc = chunk2(cc + s, acc)
            return acc

        acc = lax.fori_loop(0, 2 * NQUAD, pair, x)
        acc = chunk2(TAILC0, acc)
        return chunk2(TAILC, acc)

    acc = lax.fori_loop(0, 5, row2, acc)
    accv[pl.ds(0, 16)] = acc
    pltpu.sync_copy(accv, out_hbm.at[pl.ds(w * 16, 16)])


def _tc_reduce_body(x_ref, o_ref):
    o_ref[...] = jnp.max(x_ref[...], axis=0, keepdims=True)


_tc_reduce = pl.pallas_call(
    _tc_reduce_body,
    out_shape=jax.ShapeDtypeStruct((1,), jnp.float32),
)


def _col_masks():
    jj = np.arange(W_OUT)
    ctj = (jj * W_IN) // W_OUT
    cpj = np.clip((W_IN * jj + (W_IN // 2)) // W_OUT, 0, W_IN - 1)
    packed = np.zeros(2 * 856, np.float32)
    np.maximum.at(packed[:W_IN], ctj, (cpj == ctj).astype(np.float32))
    np.maximum.at(packed[856:856 + W_IN], ctj,
                  (cpj == ctj + 1).astype(np.float32))
    return packed


_COL_MASKS = _col_masks()


def kernel(mask):
    partial = _sc_any_diff(mask, _COL_MASKS)
    red = _tc_reduce(partial)
    return red[0] > 0.0
